# baseline (speedup 1.0000x reference)
import functools

import jax
import jax.numpy as jnp
import numpy as np
from jax import lax
from jax.experimental import pallas as pl
from jax.experimental.pallas import tpu as pltpu
from jax.experimental.pallas import tpu_sc as plsc

BATCH = 16384
HIST = 200
EMBED_DIM = 6
NTOK = BATCH * HIST
NC, NS = 2, 16
NW = NC * NS
TPW = NTOK // NW
CHUNK = 2048
NCHUNK = TPW // CHUNK
GROUP = 16
L = 16

IDX_ROWS = NTOK // 128
OUT_ROWS = NTOK * EMBED_DIM // 128
CH_IROWS = CHUNK // 128
CH_OROWS = CHUNK * EMBED_DIM // 128

_lane = np.arange(L * EMBED_DIM, dtype=np.int32)
_PATS = np.concatenate([
    (_lane // EMBED_DIM).reshape(EMBED_DIM, L),
    (_lane % EMBED_DIM).reshape(EMBED_DIM, L),
], axis=0)


def _permute(vec, idx):
    return lax.gather(
        vec, idx[:, None],
        dimension_numbers=lax.GatherDimensionNumbers(
            offset_dims=(), collapsed_slice_dims=(0,), start_index_map=(0,)),
        slice_sizes=(1,),
        mode=lax.GatherScatterMode.PROMISE_IN_BOUNDS)


def _sc_lookup(idx2d, tab_pad, pats):
    mesh = plsc.VectorSubcoreMesh(core_axis_name="c", subcore_axis_name="s")

    @functools.partial(
        pl.kernel,
        mesh=mesh,
        out_type=jax.ShapeDtypeStruct((OUT_ROWS, 128), jnp.float32),
        scratch_types=[
            pltpu.VMEM((CH_IROWS, 128), jnp.int32),
            pltpu.VMEM((CH_OROWS, 128), jnp.float32),
            pltpu.VMEM((2 * L,), jnp.float32),
            pltpu.VMEM(_PATS.shape, jnp.int32),
        ],
    )
    def k(idx_hbm, tab_hbm, pats_hbm, out_hbm, idx_v, out_v, tab_v, pats_v):
        wid = lax.axis_index("s") * NC + lax.axis_index("c")

        pltpu.sync_copy(tab_hbm, tab_v)
        pltpu.sync_copy(pats_hbm, pats_v)

        t0 = tab_v[pl.ds(0, L)]
        t1 = tab_v[pl.ds(L, L)]
        gpat, w0, dw = [], [], []
        for v in range(EMBED_DIM):
            gpat.append(pats_v[v, :])
            cp = pats_v[EMBED_DIM + v, :]
            r0 = _permute(t0, cp)
            r1 = _permute(t1, cp)
            w0.append(r0)
            dw.append(r1 - r0)

        def row_body(r, carry):
            for j in range(8):
                iv = idx_v[r, pl.ds(j * GROUP, GROUP)]
                for v in range(EMBED_DIM):
                    q = j * EMBED_DIM + v
                    pv = _permute(iv, gpat[v])
                    ov = w0[v] + pv.astype(jnp.float32) * dw[v]
                    out_v[r * EMBED_DIM + q // 8, pl.ds((q % 8) * L, L)] = ov
            return carry

        def chunk_body(ch, carry):
            irow = wid * (TPW // 128) + ch * CH_IROWS
            pltpu.sync_copy(idx_hbm.at[pl.ds(irow, CH_IROWS)], idx_v)
            lax.fori_loop(0, CH_IROWS, row_body, 0)
            pltpu.sync_copy(out_v, out_hbm.at[pl.ds(irow * EMBED_DIM,
                                                    CH_OROWS)])
            return carry

        lax.fori_loop(0, NCHUNK, chunk_body, 0)

    return k(idx2d, tab_pad, pats)


def kernel(indices, table):
    idx2d = indices.reshape(IDX_ROWS, 128).astype(jnp.int32)
    tab_pad = jnp.zeros((2, L), jnp.float32).at[:, :EMBED_DIM].set(table).reshape(-1)
    pats = jnp.asarray(_PATS)
    out2d = _sc_lookup(idx2d, tab_pad, pats)
    return out2d.reshape(BATCH, HIST, EMBED_DIM)


# speedup vs baseline: 12.1664x; 12.1664x over previous
import functools

import jax
import jax.numpy as jnp
import numpy as np
from jax import lax
from jax.experimental import pallas as pl
from jax.experimental.pallas import tpu as pltpu
from jax.experimental.pallas import tpu_sc as plsc

BATCH = 16384
HIST = 200
EMBED_DIM = 6
NTOK = BATCH * HIST
NC, NS = 2, 16
NW = NC * NS
L = 16

IDX_ROWS = NTOK // 128
RPW = IDX_ROWS // NW
CH_ROWS = 16
NCHUNK = RPW // CH_ROWS

_PATS = np.tile(np.arange(EMBED_DIM, dtype=np.int32)[:, None], (1, L))


def _permute(vec, idx):
    return lax.gather(
        vec, idx[:, None],
        dimension_numbers=lax.GatherDimensionNumbers(
            offset_dims=(), collapsed_slice_dims=(0,), start_index_map=(0,)),
        slice_sizes=(1,),
        mode=lax.GatherScatterMode.PROMISE_IN_BOUNDS)


def _sc_lookup(idx2d, tab_pad, pats):
    mesh = plsc.VectorSubcoreMesh(core_axis_name="c", subcore_axis_name="s")

    @functools.partial(
        pl.kernel,
        mesh=mesh,
        out_type=jax.ShapeDtypeStruct((EMBED_DIM * IDX_ROWS, 128),
                                      jnp.float32),
        scratch_types=[
            pltpu.VMEM((CH_ROWS, 128), jnp.int32),
            pltpu.VMEM((EMBED_DIM * CH_ROWS, 128), jnp.float32),
            pltpu.VMEM((2 * L,), jnp.float32),
            pltpu.VMEM(_PATS.shape, jnp.int32),
        ],
    )
    def k(idx_hbm, tab_hbm, pats_hbm, out_hbm, idx_v, out_v, tab_v, pats_v):
        wid = lax.axis_index("s") * NC + lax.axis_index("c")
        rbase = wid * RPW

        pltpu.sync_copy(tab_hbm, tab_v)
        pltpu.sync_copy(pats_hbm, pats_v)

        t0 = tab_v[pl.ds(0, L)]
        t1 = tab_v[pl.ds(L, L)]
        w0 = [_permute(t0, pats_v[c, :]) for c in range(EMBED_DIM)]
        w1 = [_permute(t1, pats_v[c, :]) for c in range(EMBED_DIM)]

        def row_body(r, carry):
            for j in range(8):
                iv = idx_v[r, pl.ds(j * L, L)]
                m = iv == 1
                for c in range(EMBED_DIM):
                    out_v[c * CH_ROWS + r, pl.ds(j * L, L)] = (
                        jnp.where(m, w1[c], w0[c]))
            return carry

        def chunk_body(ch, carry):
            r0 = rbase + ch * CH_ROWS
            pltpu.sync_copy(idx_hbm.at[pl.ds(r0, CH_ROWS)], idx_v)
            lax.fori_loop(0, CH_ROWS, row_body, 0)
            for c in range(EMBED_DIM):
                pltpu.sync_copy(
                    out_v.at[pl.ds(c * CH_ROWS, CH_ROWS)],
                    out_hbm.at[pl.ds(c * IDX_ROWS + r0, CH_ROWS)])
            return carry

        lax.fori_loop(0, NCHUNK, chunk_body, 0)

    return k(idx2d, tab_pad, pats)


def kernel(indices, table):
    idx2d = indices.astype(jnp.int32).T.reshape(IDX_ROWS, 128)
    tab_pad = jnp.zeros((2, L), jnp.float32).at[:, :EMBED_DIM].set(table).reshape(-1)
    pats = jnp.asarray(_PATS)
    out2d = _sc_lookup(idx2d, tab_pad, pats)
    return out2d.reshape(EMBED_DIM, HIST, BATCH).transpose(2, 1, 0)


# speedup vs baseline: 30.4470x; 2.5026x over previous
import functools

import jax
import jax.numpy as jnp
import numpy as np
from jax import lax
from jax.experimental import pallas as pl
from jax.experimental.pallas import tpu as pltpu
from jax.experimental.pallas import tpu_sc as plsc

BATCH = 16384
HIST = 200
EMBED_DIM = 6
NC, NS = 2, 16
NW = NC * NS
L = 16

CW = 2048
CPR = BATCH // CW
NCHUNKS = HIST * CPR
CPW = NCHUNKS // NW

_PATS = np.tile(np.arange(EMBED_DIM, dtype=np.int32)[:, None], (1, L))


def _permute(vec, idx):
    return lax.gather(
        vec, idx[:, None],
        dimension_numbers=lax.GatherDimensionNumbers(
            offset_dims=(), collapsed_slice_dims=(0,), start_index_map=(0,)),
        slice_sizes=(1,),
        mode=lax.GatherScatterMode.PROMISE_IN_BOUNDS)


def _sc_lookup(idx_t, tab_pad, pats):
    mesh = plsc.VectorSubcoreMesh(core_axis_name="c", subcore_axis_name="s")

    @functools.partial(
        pl.kernel,
        mesh=mesh,
        out_type=jax.ShapeDtypeStruct((EMBED_DIM, HIST, BATCH), jnp.float32),
        scratch_types=[
            pltpu.VMEM((2, CW), jnp.int32),
            pltpu.VMEM((2, EMBED_DIM, CW), jnp.float32),
            pltpu.VMEM((2 * L,), jnp.float32),
            pltpu.VMEM(_PATS.shape, jnp.int32),
            pltpu.SemaphoreType.DMA,
            pltpu.SemaphoreType.DMA,
            pltpu.SemaphoreType.DMA,
            pltpu.SemaphoreType.DMA,
        ],
    )
    def k(idx_hbm, tab_hbm, pats_hbm, out_hbm, idx_v, out_v, tab_v, pats_v,
          isem0, isem1, osem0, osem1):
        isem = [isem0, isem1]
        osem = [osem0, osem1]
        wid = lax.axis_index("s") * NC + lax.axis_index("c")
        k0 = wid * CPW

        pltpu.sync_copy(tab_hbm, tab_v)
        pltpu.sync_copy(pats_hbm, pats_v)

        t0 = tab_v[pl.ds(0, L)]
        t1 = tab_v[pl.ds(L, L)]
        w0 = [_permute(t0, pats_v[c, :]) for c in range(EMBED_DIM)]
        w1 = [_permute(t1, pats_v[c, :]) for c in range(EMBED_DIM)]

        def in_copy(kk, b):
            t = kk // CPR
            col = (kk % CPR) * CW
            return pltpu.make_async_copy(
                idx_hbm.at[t, pl.ds(col, CW)], idx_v.at[b], isem[b])

        def out_copy(kk, b, c):
            t = kk // CPR
            col = (kk % CPR) * CW
            return pltpu.make_async_copy(
                out_v.at[b, c], out_hbm.at[c, t, pl.ds(col, CW)], osem[b])

        in_copy(k0, 0).start()
        in_copy(k0 + 1, 1).start()

        def step_body(s, carry):
            for b in range(2):
                kk = k0 + s * 2 + b
                in_copy(kk, b).wait()

                @pl.when(s > 0)
                def _wait_out():
                    for c in range(EMBED_DIM):
                        out_copy(kk, b, c).wait()

                def vec_body(j, c2):
                    iv = idx_v[b, pl.ds(j * L, L)]
                    m = iv == 1
                    for c in range(EMBED_DIM):
                        out_v[b, c, pl.ds(j * L, L)] = jnp.where(m, w1[c], w0[c])
                    return c2

                lax.fori_loop(0, CW // L, vec_body, 0)

                for c in range(EMBED_DIM):
                    out_copy(kk, b, c).start()

                @pl.when(s * 2 + b + 2 < CPW)
                def _prefetch():
                    in_copy(kk + 2, b).start()
            return carry

        lax.fori_loop(0, CPW // 2, step_body, 0)

        for b in range(2):
            kk = k0 + CPW - 2 + b
            for c in range(EMBED_DIM):
                out_copy(kk, b, c).wait()

    return k(idx_t, tab_pad, pats)


def kernel(indices, table):
    idx_t = indices.astype(jnp.int32).T
    tab_pad = jnp.zeros((2, L), jnp.float32).at[:, :EMBED_DIM].set(table).reshape(-1)
    pats = jnp.asarray(_PATS)
    planes = _sc_lookup(idx_t, tab_pad, pats)
    return planes.transpose(2, 1, 0)
